# baseline (speedup 1.0000x reference)
import jax
import jax.numpy as jnp
from jax import lax
from jax.experimental import pallas as pl
from jax.experimental.pallas import tpu as pltpu

B, S, H = 1, 2048, 1024
I = 2048
E = 8
V = 100000
EI = I // E
T = B * S
NT = 2
T2 = T // NT


def _dense_body(tid_ref, x_ref, g_ref, u_ref, d_ref, o_ref, xbf_ref, int_ref):
    e = pl.program_id(1)

    @pl.when(e == 0)
    def _():
        xbf_ref[...] = x_ref[...].astype(jnp.bfloat16)

    tid = jnp.clip(tid_ref[...], 0, V - 1)
    eid = lax.rem(tid, E)
    mask = eid == e
    x = xbf_ref[...]
    gw = g_ref[0].astype(jnp.bfloat16)
    uw = u_ref[0].astype(jnp.bfloat16)
    g = jnp.dot(x, gw, preferred_element_type=jnp.float32)
    u = jnp.dot(x, uw, preferred_element_type=jnp.float32)
    inter = jnp.where(mask, g * jax.nn.sigmoid(g) * u, 0.0)
    int_ref[:, pl.ds(e * EI, EI)] = inter.astype(jnp.bfloat16)

    @pl.when(e == E - 1)
    def _():
        dw = d_ref[...].astype(jnp.bfloat16)
        o_ref[...] = jnp.dot(int_ref[...], dw,
                             preferred_element_type=jnp.float32)


def kernel(hidden_states, token_ids, mu, gate_proj, up_proj, down_proj, mu_w, token_to_expert):
    x = hidden_states.reshape(T, H)
    tid2d = token_ids.reshape(T, 1)
    down_all = down_proj.reshape(I, H)
    out = pl.pallas_call(
        _dense_body,
        grid=(NT, E),
        in_specs=[
            pl.BlockSpec((T2, 1), lambda t, e: (t, 0)),
            pl.BlockSpec((T2, H), lambda t, e: (t, 0)),
            pl.BlockSpec((1, H, EI), lambda t, e: (e, 0, 0)),
            pl.BlockSpec((1, H, EI), lambda t, e: (e, 0, 0)),
            pl.BlockSpec((I, H), lambda t, e: (0, 0)),
        ],
        out_specs=pl.BlockSpec((T2, H), lambda t, e: (t, 0)),
        out_shape=jax.ShapeDtypeStruct((T, H), jnp.float32),
        scratch_shapes=[
            pltpu.VMEM((T2, H), jnp.bfloat16),
            pltpu.VMEM((T2, I), jnp.bfloat16),
        ],
    )(tid2d, x, gate_proj, up_proj, down_all)
    return out.reshape(B, S, H)


# speedup vs baseline: 1.0335x; 1.0335x over previous
import jax
import jax.numpy as jnp
from jax import lax
from jax.experimental import pallas as pl
from jax.experimental.pallas import tpu as pltpu

B, S, H = 1, 2048, 1024
I = 2048
E = 8
V = 100000
EI = I // E
T = B * S
NT = 2
T2 = T // NT


def _dense_body(tid_ref, x_ref, g_ref, u_ref, d_ref, o_ref,
                xbf_ref, int_ref, gus_ref, ds_ref):
    t = pl.program_id(0)
    e = pl.program_id(1)

    @pl.when(e == 0)
    def _():
        xbf_ref[...] = x_ref[...].astype(jnp.bfloat16)

    @pl.when(t == 0)
    def _():
        gus_ref[e, :, :EI] = g_ref[0].astype(jnp.bfloat16)
        gus_ref[e, :, EI:] = u_ref[0].astype(jnp.bfloat16)
        ds_ref[pl.ds(e * EI, EI), :] = d_ref[0].astype(jnp.bfloat16)

    tid = jnp.clip(tid_ref[...], 0, V - 1)
    eid = lax.rem(tid, E)
    mask = eid == e
    gu = jnp.dot(xbf_ref[...], gus_ref[e],
                 preferred_element_type=jnp.float32)
    g = gu[:, :EI]
    u = gu[:, EI:]
    inter = jnp.where(mask, g * jax.nn.sigmoid(g) * u, 0.0)
    int_ref[:, pl.ds(e * EI, EI)] = inter.astype(jnp.bfloat16)

    @pl.when(e == E - 1)
    def _():
        o_ref[...] = jnp.dot(int_ref[...], ds_ref[...],
                             preferred_element_type=jnp.float32)


def kernel(hidden_states, token_ids, mu, gate_proj, up_proj, down_proj, mu_w, token_to_expert):
    x = hidden_states.reshape(T, H)
    tid2d = token_ids.reshape(T, 1)
    widx = lambda t, e: (e + t * (E - 1 - e), 0, 0)
    out = pl.pallas_call(
        _dense_body,
        grid=(NT, E),
        in_specs=[
            pl.BlockSpec((T2, 1), lambda t, e: (t, 0)),
            pl.BlockSpec((T2, H), lambda t, e: (t, 0)),
            pl.BlockSpec((1, H, EI), widx),
            pl.BlockSpec((1, H, EI), widx),
            pl.BlockSpec((1, EI, H), widx),
        ],
        out_specs=pl.BlockSpec((T2, H), lambda t, e: (t, 0)),
        out_shape=jax.ShapeDtypeStruct((T, H), jnp.float32),
        scratch_shapes=[
            pltpu.VMEM((T2, H), jnp.bfloat16),
            pltpu.VMEM((T2, I), jnp.bfloat16),
            pltpu.VMEM((E, H, 2 * EI), jnp.bfloat16),
            pltpu.VMEM((I, H), jnp.bfloat16),
        ],
    )(tid2d, x, gate_proj, up_proj, down_proj)
    return out.reshape(B, S, H)
